# Initial kernel scaffold; baseline (speedup 1.0000x reference)
import jax
import jax.numpy as jnp
from jax.experimental import pallas as pl


def kernel(token_ids, weight):
    raise NotImplementedError("write your pallas kernel here")


# speedup vs baseline: 1.6860x; 1.6860x over previous
import functools

import jax
import jax.numpy as jnp
from jax import lax
from jax.experimental import pallas as pl
from jax.experimental.pallas import tpu as pltpu
from jax.experimental.pallas import tpu_sc as plsc

DIM = 64
CHUNK = 128
N_WORKERS = 32


@functools.partial(jax.jit, static_argnums=(2,))
def _gather_rows(idx, weight, n_chunks):
    @functools.partial(
        pl.kernel,
        out_type=jax.ShapeDtypeStruct((N_WORKERS * n_chunks * CHUNK, DIM), jnp.float32),
        mesh=plsc.VectorSubcoreMesh(core_axis_name="c", subcore_axis_name="s"),
        scratch_types=[
            pltpu.VMEM((n_chunks, CHUNK), jnp.int32),
            pltpu.VMEM((CHUNK, DIM), jnp.float32),
            pltpu.SemaphoreType.DMA,
        ],
        compiler_params=pltpu.CompilerParams(use_tc_tiling_on_sc=False),
    )
    def k(idx_hbm, table_hbm, out_hbm, idx_v, rows_v, gsem):
        wid = lax.axis_index("s") * 2 + lax.axis_index("c")
        pltpu.sync_copy(idx_hbm.at[wid], idx_v)
        base = wid * (n_chunks * CHUNK)

        @pl.loop(0, n_chunks)
        def _(j):
            pltpu.async_copy(table_hbm.at[idx_v.at[j]], rows_v, gsem).wait()
            pltpu.sync_copy(rows_v, out_hbm.at[pl.ds(base + j * CHUNK, CHUNK)])

    return k(idx, weight)


def kernel(token_ids, weight):
    B, H = token_ids.shape
    total = B * H
    flat = token_ids.reshape(total).astype(jnp.int32)
    per = N_WORKERS * CHUNK
    padded = ((total + per - 1) // per) * per
    if padded != total:
        flat = jnp.pad(flat, (0, padded - total))
    n_chunks = padded // per
    idx = flat.reshape(N_WORKERS, n_chunks, CHUNK)
    out = _gather_rows(idx, weight, n_chunks)
    return out[:total].reshape(B, H, DIM)


# speedup vs baseline: 1.8707x; 1.1096x over previous
import functools

import jax
import jax.numpy as jnp
from jax import lax
from jax.experimental import pallas as pl
from jax.experimental.pallas import tpu as pltpu
from jax.experimental.pallas import tpu_sc as plsc

DIM = 64
CHUNK = 128
K = 4
N_WORKERS = 32


@functools.partial(jax.jit, static_argnums=(2,))
def _gather_rows(idx, weight, n_chunks):
    n_super = n_chunks // K
    assert n_chunks == n_super * K and n_super % 2 == 0 and n_super >= 4

    @functools.partial(
        pl.kernel,
        out_type=jax.ShapeDtypeStruct((N_WORKERS * n_chunks * CHUNK, DIM), jnp.float32),
        mesh=plsc.VectorSubcoreMesh(core_axis_name="c", subcore_axis_name="s"),
        scratch_types=[
            pltpu.VMEM((n_chunks, CHUNK), jnp.int32),
            pltpu.VMEM((2 * K, CHUNK, DIM), jnp.float32),
            pltpu.SemaphoreType.DMA,
            pltpu.SemaphoreType.DMA,
            pltpu.SemaphoreType.DMA,
            pltpu.SemaphoreType.DMA,
        ],
        compiler_params=pltpu.CompilerParams(use_tc_tiling_on_sc=False),
    )
    def k(idx_hbm, table_hbm, out_hbm, idx_v, rows, gsem0, gsem1, osem0, osem1):
        wid = lax.axis_index("s") * 2 + lax.axis_index("c")
        pltpu.sync_copy(idx_hbm.at[wid], idx_v)
        base = wid * (n_chunks * CHUNK)
        gsems = (gsem0, gsem1)
        osems = (osem0, osem1)

        def out_slice(j):
            return out_hbm.at[pl.ds(base + j * CHUNK, CHUNK)]

        def fire_gathers(g, grp):
            for b in range(K):
                pltpu.async_copy(
                    table_hbm.at[idx_v.at[g * K + b]], rows.at[grp * K + b], gsems[grp]
                )

        def drain_gathers(g, grp):
            for b in range(K):
                pltpu.make_async_copy(
                    table_hbm.at[idx_v.at[g * K + b]], rows.at[grp * K + b], gsems[grp]
                ).wait()

        def fire_outs(g, grp):
            for b in range(K):
                pltpu.async_copy(rows.at[grp * K + b], out_slice(g * K + b), osems[grp])

        def drain_outs(g, grp):
            for b in range(K):
                pltpu.make_async_copy(
                    rows.at[grp * K + b], out_slice(g * K + b), osems[grp]
                ).wait()

        def step(g, cur):
            nxt = 1 - cur
            drain_gathers(g, cur)
            fire_outs(g, cur)
            drain_outs(g - 1, nxt)
            fire_gathers(g + 1, nxt)

        fire_gathers(0, 0)
        drain_gathers(0, 0)
        fire_outs(0, 0)
        fire_gathers(1, 1)

        @pl.loop(0, (n_super - 2) // 2)
        def _(p):
            g = 1 + 2 * p
            step(g, 1)
            step(g + 1, 0)

        g_last = n_super - 1
        drain_gathers(g_last, 1)
        fire_outs(g_last, 1)
        drain_outs(g_last - 1, 0)
        drain_outs(g_last, 1)

    return k(idx, weight)


def kernel(token_ids, weight):
    B, H = token_ids.shape
    total = B * H
    flat = token_ids.reshape(total).astype(jnp.int32)
    per = N_WORKERS * CHUNK * K * 2
    padded = ((total + per - 1) // per) * per
    if padded != total:
        flat = jnp.pad(flat, (0, padded - total))
    n_chunks = padded // (N_WORKERS * CHUNK)
    idx = flat.reshape(N_WORKERS, n_chunks, CHUNK)
    out = _gather_rows(idx, weight, n_chunks)
    return out[:total].reshape(B, H, DIM)


# speedup vs baseline: 1.8712x; 1.0002x over previous
import functools

import jax
import jax.numpy as jnp
from jax import lax
from jax.experimental import pallas as pl
from jax.experimental.pallas import tpu as pltpu
from jax.experimental.pallas import tpu_sc as plsc

DIM = 64
SUPER = 512
N_WORKERS = 32


@functools.partial(jax.jit, static_argnums=(2,))
def _gather_rows(idx, weight, n_super):
    assert n_super % 2 == 0 and n_super >= 4

    @functools.partial(
        pl.kernel,
        out_type=jax.ShapeDtypeStruct((N_WORKERS * n_super * SUPER, DIM), jnp.float32),
        mesh=plsc.VectorSubcoreMesh(core_axis_name="c", subcore_axis_name="s"),
        scratch_types=[
            pltpu.VMEM((n_super, SUPER), jnp.int32),
            pltpu.VMEM((2, SUPER, DIM), jnp.float32),
            pltpu.SemaphoreType.DMA,
            pltpu.SemaphoreType.DMA,
            pltpu.SemaphoreType.DMA,
            pltpu.SemaphoreType.DMA,
        ],
        compiler_params=pltpu.CompilerParams(use_tc_tiling_on_sc=False),
    )
    def k(idx_hbm, table_hbm, out_hbm, idx_v, rows, gsem0, gsem1, osem0, osem1):
        wid = lax.axis_index("s") * 2 + lax.axis_index("c")
        pltpu.sync_copy(idx_hbm.at[wid], idx_v)
        base = wid * (n_super * SUPER)
        gsems = (gsem0, gsem1)
        osems = (osem0, osem1)

        def out_slice(g):
            return out_hbm.at[pl.ds(base + g * SUPER, SUPER)]

        def fire_gather(g, grp):
            pltpu.async_copy(table_hbm.at[idx_v.at[g]], rows.at[grp], gsems[grp])

        def drain_gather(g, grp):
            pltpu.make_async_copy(
                table_hbm.at[idx_v.at[g]], rows.at[grp], gsems[grp]
            ).wait()

        def fire_out(g, grp):
            pltpu.async_copy(rows.at[grp], out_slice(g), osems[grp])

        def drain_out(g, grp):
            pltpu.make_async_copy(rows.at[grp], out_slice(g), osems[grp]).wait()

        def step(g, cur):
            nxt = 1 - cur
            drain_gather(g, cur)
            fire_out(g, cur)
            drain_out(g - 1, nxt)
            fire_gather(g + 1, nxt)

        fire_gather(0, 0)
        drain_gather(0, 0)
        fire_out(0, 0)
        fire_gather(1, 1)

        @pl.loop(0, (n_super - 2) // 2)
        def _(p):
            g = 1 + 2 * p
            step(g, 1)
            step(g + 1, 0)

        g_last = n_super - 1
        drain_gather(g_last, 1)
        fire_out(g_last, 1)
        drain_out(g_last - 1, 0)
        drain_out(g_last, 1)

    return k(idx, weight)


def kernel(token_ids, weight):
    B, H = token_ids.shape
    total = B * H
    flat = token_ids.reshape(total).astype(jnp.int32)
    per = N_WORKERS * SUPER * 2
    padded = ((total + per - 1) // per) * per
    if padded != total:
        flat = jnp.pad(flat, (0, padded - total))
    n_super = padded // (N_WORKERS * SUPER)
    idx = flat.reshape(N_WORKERS, n_super, SUPER)
    out = _gather_rows(idx, weight, n_super)
    return out[:total].reshape(B, H, DIM)
